# baseline (speedup 1.0000x reference)
import functools

import jax
import jax.numpy as jnp
from jax import lax
from jax.experimental import pallas as pl
from jax.experimental.pallas import tpu as pltpu
from jax.experimental.pallas import tpu_sc as plsc

F32 = jnp.float32
I32 = jnp.int32

N = 10000
E = 320000
NP = 10240
G = 128
DH = 256
DHH = 128
VSZ = 30
OD = 30954
ODP = 31232

EC = 128
ROWS = 2560
EP = ROWS * EC
RPT_MSG = ROWS // 16
RPT_DEG = ROWS // 32
TROWS = NP // 16
NB = 1280
OB = 512

_sc_mesh = plsc.VectorSubcoreMesh(
    core_axis_name="c", subcore_axis_name="s", num_cores=2, num_subcores=16)


@functools.partial(
    pl.kernel,
    out_type=jax.ShapeDtypeStruct((2, NP), F32),
    mesh=_sc_mesh,
    scratch_types=[
        pltpu.VMEM((RPT_DEG, EC), I32),
        pltpu.VMEM((EC,), F32),
        pltpu.VMEM((TROWS,), F32),
        pltpu.VMEM_SHARED((NP,), F32),
    ],
)
def _deg_kernel(dst_hbm, out_hbm, idx_v, ones_v, buf_v, acc_sh):
    c = lax.axis_index("c")
    s = lax.axis_index("s")
    w = s * 2 + c

    def fill_ones(i, _):
        ones_v[pl.ds(i * 16, 16)] = jnp.full((16,), 1.0, F32)
        return 0
    lax.fori_loop(0, EC // 16, fill_ones, 0)

    def fill_zero(i, _):
        buf_v[pl.ds(i * 16, 16)] = jnp.zeros((16,), F32)
        return 0
    lax.fori_loop(0, TROWS // 16, fill_zero, 0)
    pltpu.sync_copy(buf_v, acc_sh.at[pl.ds(s * TROWS, TROWS)])
    plsc.subcore_barrier()

    pltpu.sync_copy(dst_hbm.at[pl.ds(w * RPT_DEG, RPT_DEG)], idx_v)

    def chunk(j, _):
        pltpu.sync_copy(ones_v, acc_sh.at[idx_v.at[j]], add=True)
        return 0
    lax.fori_loop(0, RPT_DEG, chunk, 0)
    plsc.subcore_barrier()

    pltpu.sync_copy(acc_sh.at[pl.ds(s * TROWS, TROWS)], buf_v)
    pltpu.sync_copy(buf_v, out_hbm.at[c].at[pl.ds(s * TROWS, TROWS)])


@functools.partial(
    pl.kernel,
    out_type=jax.ShapeDtypeStruct((2, NP, DHH), F32),
    mesh=_sc_mesh,
    scratch_types=[
        pltpu.VMEM((2, 8, EC), I32),
        pltpu.VMEM((2, 8, EC), I32),
        pltpu.VMEM((2, EC, DHH), F32),
        pltpu.VMEM_SHARED((NP, DHH), F32),
        pltpu.SemaphoreType.DMA,
        pltpu.SemaphoreType.DMA,
    ],
)
def _msg_kernel(hs_hbm, src_hbm, dst_hbm, out_hbm, isrc_v, idst_v, rows_v,
                acc_sh, gsem, ssem):
    c = lax.axis_index("c")
    s = lax.axis_index("s")

    def fill_zero(i, _):
        rows_v[0, i // 8, pl.ds((i % 8) * 16, 16)] = jnp.zeros((16,), F32)
        return 0
    lax.fori_loop(0, EC * DHH // 16, fill_zero, 0)

    def zero_cp(t, _):
        pltpu.sync_copy(rows_v.at[0], acc_sh.at[pl.ds(s * TROWS + t * EC, EC)])
        return 0
    lax.fori_loop(0, TROWS // EC, zero_cp, 0)
    plsc.subcore_barrier()

    def load_idx(g):
        gb = lax.rem(g, 2)
        r0 = s * RPT_MSG + g * 8
        pltpu.sync_copy(src_hbm.at[pl.ds(r0, 8)], isrc_v.at[gb])
        pltpu.sync_copy(dst_hbm.at[pl.ds(r0, 8)], idst_v.at[gb])

    def g_half(j, b, h):
        g = j // 8
        idx = isrc_v.at[lax.rem(g, 2), lax.rem(j, 8), pl.ds(h * 64, 64)]
        return pltpu.make_async_copy(
            hs_hbm.at[c].at[idx], rows_v.at[b].at[pl.ds(h * 64, 64)], gsem)

    def s_desc(j, b):
        g = j // 8
        return pltpu.make_async_copy(
            rows_v.at[b],
            acc_sh.at[idst_v.at[lax.rem(g, 2), lax.rem(j, 8)]], ssem)

    load_idx(0)
    g_half(0, 0, 0).start()
    g_half(0, 0, 1).start()

    def chunk(j, _):
        b = lax.rem(j, 2)
        bn = lax.rem(j + 1, 2)
        k = lax.rem(j, 8)
        g_half(j, b, 0).wait()
        g_half(j, b, 1).wait()

        @pl.when(j >= 1)
        def _():
            s_desc(j - 1, bn).wait()

        @pl.when(jnp.logical_and(k == 7, j + 1 < RPT_MSG))
        def _():
            load_idx(j // 8 + 1)

        @pl.when(j + 1 < RPT_MSG)
        def _():
            g_half(j + 1, bn, 0).start()
            g_half(j + 1, bn, 1).start()

        g = j // 8
        pltpu.async_copy(
            rows_v.at[b],
            acc_sh.at[idst_v.at[lax.rem(g, 2), lax.rem(j, 8)]], ssem, add=True)
        return 0
    lax.fori_loop(0, RPT_MSG, chunk, 0)
    s_desc(RPT_MSG - 1, lax.rem(RPT_MSG - 1, 2)).wait()
    plsc.subcore_barrier()

    def drain(t, _):
        r0 = s * TROWS + t * EC
        pltpu.sync_copy(acc_sh.at[pl.ds(r0, EC)], rows_v.at[0])
        pltpu.sync_copy(rows_v.at[0], out_hbm.at[c].at[pl.ds(r0, EC)])
        return 0
    lax.fori_loop(0, TROWS // EC, drain, 0)


_PREC = lax.Precision.HIGHEST


def _dinv(dega, degb):
    return lax.rsqrt(dega + degb + 1.0)


def _hs1_body(xf_ref, dega_ref, degb_ref, emb_ref, w1_ref, out_ref):
    dinv = _dinv(dega_ref[...], degb_ref[...])
    ew = jnp.dot(emb_ref[...], w1_ref[...],
                 preferred_element_type=F32, precision=_PREC)
    ids = lax.broadcasted_iota(I32, (1, 32), 1).astype(F32)
    oh = (xf_ref[...] == ids).astype(F32)
    hs = jnp.dot(oh, ew, preferred_element_type=F32,
                 precision=_PREC) * dinv
    out_ref[0] = hs[:, :DHH]
    out_ref[1] = hs[:, DHH:]


def _layer2_body(s_ref, h_ref, dega_ref, degb_ref, b1_ref, w2_ref, out_ref):
    dinv = _dinv(dega_ref[...], degb_ref[...])
    t = jnp.concatenate([s_ref[0] + h_ref[0], s_ref[1] + h_ref[1]], axis=1)
    h1 = jnp.maximum(dinv * t + b1_ref[...], 0.0)
    hs = jnp.dot(h1, w2_ref[...], preferred_element_type=F32,
                 precision=_PREC) * dinv
    out_ref[0] = hs[:, :DHH]
    out_ref[1] = hs[:, DHH:]


def _pool_body(s_ref, h_ref, dega_ref, degb_ref, b2_ref, batch_ref,
               gsum_ref, cnt_ref):
    i = pl.program_id(0)
    dinv = _dinv(dega_ref[...], degb_ref[...])
    t = jnp.concatenate([s_ref[0] + h_ref[0], s_ref[1] + h_ref[1]], axis=1)
    h2 = jnp.maximum(dinv * t + b2_ref[...], 0.0)
    ids = lax.broadcasted_iota(I32, (1, G), 1).astype(F32)
    oh = (batch_ref[...] == ids).astype(F32)

    @pl.when(i == 0)
    def _():
        gsum_ref[...] = jnp.zeros_like(gsum_ref)
        cnt_ref[...] = jnp.zeros_like(cnt_ref)

    dn = (((0,), (0,)), ((), ()))
    gsum_ref[...] += lax.dot_general(oh, h2, dn, preferred_element_type=F32,
                                     precision=_PREC)
    cnt_ref[...] += lax.dot_general(oh, jnp.ones((NB, 1), F32), dn,
                                    preferred_element_type=F32,
                                    precision=_PREC)


def _cls_body(gsum_ref, cnt_ref, wc_ref, bc_ref, out_ref):
    g = gsum_ref[...] / jnp.maximum(cnt_ref[...], 1.0)
    out_ref[...] = jnp.dot(g, wc_ref[...], preferred_element_type=F32,
                           precision=_PREC) + bc_ref[...]


def _hs1_call(xf, dega, degb, embp, w1):
    return pl.pallas_call(
        _hs1_body,
        grid=(NP // NB,),
        in_specs=[
            pl.BlockSpec((NB, 1), lambda i: (i, 0)),
            pl.BlockSpec((NB, 1), lambda i: (i, 0)),
            pl.BlockSpec((NB, 1), lambda i: (i, 0)),
            pl.BlockSpec((32, DHH), lambda i: (0, 0)),
            pl.BlockSpec((DHH, DH), lambda i: (0, 0)),
        ],
        out_specs=pl.BlockSpec((2, NB, DHH), lambda i: (0, i, 0)),
        out_shape=jax.ShapeDtypeStruct((2, NP, DHH), F32),
    )(xf, dega, degb, embp, w1)


def _layer2_call(s1, hs1, dega, degb, b1r, w2):
    return pl.pallas_call(
        _layer2_body,
        grid=(NP // NB,),
        in_specs=[
            pl.BlockSpec((2, NB, DHH), lambda i: (0, i, 0)),
            pl.BlockSpec((2, NB, DHH), lambda i: (0, i, 0)),
            pl.BlockSpec((NB, 1), lambda i: (i, 0)),
            pl.BlockSpec((NB, 1), lambda i: (i, 0)),
            pl.BlockSpec((1, DH), lambda i: (0, 0)),
            pl.BlockSpec((DH, DH), lambda i: (0, 0)),
        ],
        out_specs=pl.BlockSpec((2, NB, DHH), lambda i: (0, i, 0)),
        out_shape=jax.ShapeDtypeStruct((2, NP, DHH), F32),
    )(s1, hs1, dega, degb, b1r, w2)


def _pool_call(s2, hs2, dega, degb, b2r, batchf):
    return pl.pallas_call(
        _pool_body,
        grid=(NP // NB,),
        in_specs=[
            pl.BlockSpec((2, NB, DHH), lambda i: (0, i, 0)),
            pl.BlockSpec((2, NB, DHH), lambda i: (0, i, 0)),
            pl.BlockSpec((NB, 1), lambda i: (i, 0)),
            pl.BlockSpec((NB, 1), lambda i: (i, 0)),
            pl.BlockSpec((1, DH), lambda i: (0, 0)),
            pl.BlockSpec((NB, 1), lambda i: (i, 0)),
        ],
        out_specs=[
            pl.BlockSpec((G, DH), lambda i: (0, 0)),
            pl.BlockSpec((G, 1), lambda i: (0, 0)),
        ],
        out_shape=[
            jax.ShapeDtypeStruct((G, DH), F32),
            jax.ShapeDtypeStruct((G, 1), F32),
        ],
    )(s2, hs2, dega, degb, b2r, batchf)


def _cls_call(gsum, cnt, wcp, bcp):
    return pl.pallas_call(
        _cls_body,
        grid=(ODP // OB,),
        in_specs=[
            pl.BlockSpec((G, DH), lambda i: (0, 0)),
            pl.BlockSpec((G, 1), lambda i: (0, 0)),
            pl.BlockSpec((DH, OB), lambda i: (0, i)),
            pl.BlockSpec((1, OB), lambda i: (0, i)),
        ],
        out_specs=pl.BlockSpec((G, OB), lambda i: (0, i)),
        out_shape=jax.ShapeDtypeStruct((G, ODP), F32),
    )(gsum, cnt, wcp, bcp)


def kernel(x, edge_index, batch, emb_table, W1, b1, W2, b2, Wc, bc):
    xf = jnp.pad(x.astype(F32), ((0, NP - N), (0, 0)),
                 constant_values=float(VSZ))
    batchf = jnp.pad(batch.astype(F32)[:, None], ((0, NP - N), (0, 0)),
                     constant_values=float(G))
    src = jnp.concatenate(
        [edge_index[0], jnp.zeros((EP - E,), I32)]).reshape(ROWS, EC)
    dst = jnp.concatenate(
        [edge_index[1], jnp.full((EP - E,), N, I32)]).reshape(ROWS, EC)
    embp = jnp.pad(emb_table, ((0, 32 - VSZ), (0, 0)))
    b1r = b1[None, :]
    b2r = b2[None, :]
    wcp = jnp.pad(Wc, ((0, 0), (0, ODP - OD)))
    bcp = jnp.pad(bc, (0, ODP - OD))[None, :]

    deg2 = _deg_kernel(dst)
    dega = deg2[0].reshape(NP, 1)
    degb = deg2[1].reshape(NP, 1)

    hs1 = _hs1_call(xf, dega, degb, embp, W1)
    s1 = _msg_kernel(hs1, src, dst)
    hs2 = _layer2_call(s1, hs1, dega, degb, b1r, W2)
    s2 = _msg_kernel(hs2, src, dst)
    gsum, cnt = _pool_call(s2, hs2, dega, degb, b2r, batchf)
    outp = _cls_call(gsum, cnt, wcp, bcp)
    return outp[:, :OD]


# speedup vs baseline: 1.0344x; 1.0344x over previous
import functools

import jax
import jax.numpy as jnp
from jax import lax
from jax.experimental import pallas as pl
from jax.experimental.pallas import tpu as pltpu
from jax.experimental.pallas import tpu_sc as plsc

F32 = jnp.float32
I32 = jnp.int32

N = 10000
E = 320000
NP = 10240
G = 128
DH = 256
DHH = 128
VSZ = 30
OD = 30954
ODP = 31232

EC = 128
ROWS = 2560
EP = ROWS * EC
RPT_MSG = ROWS // 16
RPT_DEG = ROWS // 32
TROWS = NP // 16
NB = 1280
OB = 512

_sc_mesh = plsc.VectorSubcoreMesh(
    core_axis_name="c", subcore_axis_name="s", num_cores=2, num_subcores=16)


@functools.partial(
    pl.kernel,
    out_type=jax.ShapeDtypeStruct((2, NP), F32),
    mesh=_sc_mesh,
    scratch_types=[
        pltpu.VMEM((RPT_DEG, EC), I32),
        pltpu.VMEM((EC,), F32),
        pltpu.VMEM((TROWS,), F32),
        pltpu.VMEM_SHARED((NP,), F32),
    ],
)
def _deg_kernel(dst_hbm, out_hbm, idx_v, ones_v, buf_v, acc_sh):
    c = lax.axis_index("c")
    s = lax.axis_index("s")
    w = s * 2 + c

    def fill_ones(i, _):
        ones_v[pl.ds(i * 16, 16)] = jnp.full((16,), 1.0, F32)
        return 0
    lax.fori_loop(0, EC // 16, fill_ones, 0)

    def fill_zero(i, _):
        buf_v[pl.ds(i * 16, 16)] = jnp.zeros((16,), F32)
        return 0
    lax.fori_loop(0, TROWS // 16, fill_zero, 0)
    pltpu.sync_copy(buf_v, acc_sh.at[pl.ds(s * TROWS, TROWS)])
    plsc.subcore_barrier()

    pltpu.sync_copy(dst_hbm.at[pl.ds(w * RPT_DEG, RPT_DEG)], idx_v)

    def chunk(j, _):
        pltpu.sync_copy(ones_v, acc_sh.at[idx_v.at[j]], add=True)
        return 0
    lax.fori_loop(0, RPT_DEG, chunk, 0)
    plsc.subcore_barrier()

    pltpu.sync_copy(acc_sh.at[pl.ds(s * TROWS, TROWS)], buf_v)
    pltpu.sync_copy(buf_v, out_hbm.at[c].at[pl.ds(s * TROWS, TROWS)])


@functools.partial(
    pl.kernel,
    out_type=jax.ShapeDtypeStruct((2, NP, DHH), F32),
    mesh=_sc_mesh,
    scratch_types=[
        pltpu.VMEM((2, 8, EC), I32),
        pltpu.VMEM((2, 8, EC), I32),
        pltpu.VMEM((2, EC, DHH), F32),
        pltpu.VMEM_SHARED((NP, DHH), F32),
        pltpu.SemaphoreType.DMA,
        pltpu.SemaphoreType.DMA,
    ],
)
def _msg_kernel(hs_hbm, src_hbm, dst_hbm, out_hbm, isrc_v, idst_v, rows_v,
                acc_sh, gsem, ssem):
    c = lax.axis_index("c")
    s = lax.axis_index("s")

    def fill_zero(i, _):
        rows_v[0, i // 8, pl.ds((i % 8) * 16, 16)] = jnp.zeros((16,), F32)
        return 0
    lax.fori_loop(0, EC * DHH // 16, fill_zero, 0)

    def zero_cp(t, _):
        pltpu.sync_copy(rows_v.at[0], acc_sh.at[pl.ds(s * TROWS + t * EC, EC)])
        return 0
    lax.fori_loop(0, TROWS // EC, zero_cp, 0)
    plsc.subcore_barrier()

    def load_idx(g):
        gb = lax.rem(g, 2)
        r0 = s * RPT_MSG + g * 8
        pltpu.sync_copy(src_hbm.at[pl.ds(r0, 8)], isrc_v.at[gb])
        pltpu.sync_copy(dst_hbm.at[pl.ds(r0, 8)], idst_v.at[gb])

    def g_desc(j, b):
        g = j // 8
        return pltpu.make_async_copy(
            hs_hbm.at[c].at[isrc_v.at[lax.rem(g, 2), lax.rem(j, 8)]],
            rows_v.at[b], gsem)

    def s_desc(j, b):
        g = j // 8
        return pltpu.make_async_copy(
            rows_v.at[b],
            acc_sh.at[idst_v.at[lax.rem(g, 2), lax.rem(j, 8)]], ssem)

    load_idx(0)
    g_desc(0, 0).start()

    def chunk(j, _):
        b = lax.rem(j, 2)
        bn = lax.rem(j + 1, 2)
        k = lax.rem(j, 8)
        g_desc(j, b).wait()

        @pl.when(j >= 1)
        def _():
            s_desc(j - 1, bn).wait()

        @pl.when(jnp.logical_and(k == 7, j + 1 < RPT_MSG))
        def _():
            load_idx(j // 8 + 1)

        @pl.when(j + 1 < RPT_MSG)
        def _():
            g_desc(j + 1, bn).start()

        g = j // 8
        pltpu.async_copy(
            rows_v.at[b],
            acc_sh.at[idst_v.at[lax.rem(g, 2), lax.rem(j, 8)]], ssem, add=True)
        return 0
    lax.fori_loop(0, RPT_MSG, chunk, 0)
    s_desc(RPT_MSG - 1, lax.rem(RPT_MSG - 1, 2)).wait()
    plsc.subcore_barrier()

    def drain(t, _):
        r0 = s * TROWS + t * EC
        pltpu.sync_copy(acc_sh.at[pl.ds(r0, EC)], rows_v.at[0])
        pltpu.sync_copy(rows_v.at[0], out_hbm.at[c].at[pl.ds(r0, EC)])
        return 0
    lax.fori_loop(0, TROWS // EC, drain, 0)


_PREC = lax.Precision.HIGHEST


def _dinv(dega, degb):
    return lax.rsqrt(dega + degb + 1.0)


def _hs1_body(xf_ref, dega_ref, degb_ref, emb_ref, w1_ref, out_ref):
    dinv = _dinv(dega_ref[...], degb_ref[...])
    ew = jnp.dot(emb_ref[...], w1_ref[...],
                 preferred_element_type=F32, precision=_PREC)
    ids = lax.broadcasted_iota(I32, (1, 32), 1).astype(F32)
    oh = (xf_ref[...] == ids).astype(F32)
    hs = jnp.dot(oh, ew, preferred_element_type=F32,
                 precision=_PREC) * dinv
    out_ref[0] = hs[:, :DHH]
    out_ref[1] = hs[:, DHH:]


def _layer2_body(s_ref, h_ref, dega_ref, degb_ref, b1_ref, w2_ref, out_ref):
    dinv = _dinv(dega_ref[...], degb_ref[...])
    t = jnp.concatenate([s_ref[0] + h_ref[0], s_ref[1] + h_ref[1]], axis=1)
    h1 = jnp.maximum(dinv * t + b1_ref[...], 0.0)
    hs = jnp.dot(h1, w2_ref[...], preferred_element_type=F32,
                 precision=_PREC) * dinv
    out_ref[0] = hs[:, :DHH]
    out_ref[1] = hs[:, DHH:]


def _pool_body(s_ref, h_ref, dega_ref, degb_ref, b2_ref, batch_ref,
               gsum_ref, cnt_ref):
    i = pl.program_id(0)
    dinv = _dinv(dega_ref[...], degb_ref[...])
    t = jnp.concatenate([s_ref[0] + h_ref[0], s_ref[1] + h_ref[1]], axis=1)
    h2 = jnp.maximum(dinv * t + b2_ref[...], 0.0)
    ids = lax.broadcasted_iota(I32, (1, G), 1).astype(F32)
    oh = (batch_ref[...] == ids).astype(F32)

    @pl.when(i == 0)
    def _():
        gsum_ref[...] = jnp.zeros_like(gsum_ref)
        cnt_ref[...] = jnp.zeros_like(cnt_ref)

    dn = (((0,), (0,)), ((), ()))
    gsum_ref[...] += lax.dot_general(oh, h2, dn, preferred_element_type=F32,
                                     precision=_PREC)
    cnt_ref[...] += lax.dot_general(oh, jnp.ones((NB, 1), F32), dn,
                                    preferred_element_type=F32,
                                    precision=_PREC)


def _cls_body(gsum_ref, cnt_ref, wc_ref, bc_ref, out_ref):
    g = gsum_ref[...] / jnp.maximum(cnt_ref[...], 1.0)
    out_ref[...] = jnp.dot(g, wc_ref[...], preferred_element_type=F32,
                           precision=_PREC) + bc_ref[...]


def _hs1_call(xf, dega, degb, embp, w1):
    return pl.pallas_call(
        _hs1_body,
        grid=(NP // NB,),
        in_specs=[
            pl.BlockSpec((NB, 1), lambda i: (i, 0)),
            pl.BlockSpec((NB, 1), lambda i: (i, 0)),
            pl.BlockSpec((NB, 1), lambda i: (i, 0)),
            pl.BlockSpec((32, 128), lambda i: (0, 0)),
            pl.BlockSpec((128, DH), lambda i: (0, 0)),
        ],
        out_specs=pl.BlockSpec((2, NB, DHH), lambda i: (0, i, 0)),
        out_shape=jax.ShapeDtypeStruct((2, NP, DHH), F32),
    )(xf, dega, degb, embp, w1)


def _layer2_call(s1, hs1, dega, degb, b1r, w2):
    return pl.pallas_call(
        _layer2_body,
        grid=(NP // NB,),
        in_specs=[
            pl.BlockSpec((2, NB, DHH), lambda i: (0, i, 0)),
            pl.BlockSpec((2, NB, DHH), lambda i: (0, i, 0)),
            pl.BlockSpec((NB, 1), lambda i: (i, 0)),
            pl.BlockSpec((NB, 1), lambda i: (i, 0)),
            pl.BlockSpec((1, DH), lambda i: (0, 0)),
            pl.BlockSpec((DH, DH), lambda i: (0, 0)),
        ],
        out_specs=pl.BlockSpec((2, NB, DHH), lambda i: (0, i, 0)),
        out_shape=jax.ShapeDtypeStruct((2, NP, DHH), F32),
    )(s1, hs1, dega, degb, b1r, w2)


def _pool_call(s2, hs2, dega, degb, b2r, batchf):
    return pl.pallas_call(
        _pool_body,
        grid=(NP // NB,),
        in_specs=[
            pl.BlockSpec((2, NB, DHH), lambda i: (0, i, 0)),
            pl.BlockSpec((2, NB, DHH), lambda i: (0, i, 0)),
            pl.BlockSpec((NB, 1), lambda i: (i, 0)),
            pl.BlockSpec((NB, 1), lambda i: (i, 0)),
            pl.BlockSpec((1, DH), lambda i: (0, 0)),
            pl.BlockSpec((NB, 1), lambda i: (i, 0)),
        ],
        out_specs=[
            pl.BlockSpec((G, DH), lambda i: (0, 0)),
            pl.BlockSpec((G, 1), lambda i: (0, 0)),
        ],
        out_shape=[
            jax.ShapeDtypeStruct((G, DH), F32),
            jax.ShapeDtypeStruct((G, 1), F32),
        ],
    )(s2, hs2, dega, degb, b2r, batchf)


def _cls_call(gsum, cnt, wcp, bcp):
    return pl.pallas_call(
        _cls_body,
        grid=((OD + OB - 1) // OB,),
        in_specs=[
            pl.BlockSpec((G, DH), lambda i: (0, 0)),
            pl.BlockSpec((G, 1), lambda i: (0, 0)),
            pl.BlockSpec((DH, OB), lambda i: (0, i)),
            pl.BlockSpec((1, OB), lambda i: (0, i)),
        ],
        out_specs=pl.BlockSpec((G, OB), lambda i: (0, i)),
        out_shape=jax.ShapeDtypeStruct((G, OD), F32),
    )(gsum, cnt, wcp, bcp)


def kernel(x, edge_index, batch, emb_table, W1, b1, W2, b2, Wc, bc):
    xf = jnp.pad(x.astype(F32), ((0, NP - N), (0, 0)),
                 constant_values=float(VSZ))
    batchf = jnp.pad(batch.astype(F32)[:, None], ((0, NP - N), (0, 0)),
                     constant_values=float(G))
    src = jnp.concatenate(
        [edge_index[0], jnp.zeros((EP - E,), I32)]).reshape(ROWS, EC)
    dst = jnp.concatenate(
        [edge_index[1], jnp.full((EP - E,), N, I32)]).reshape(ROWS, EC)
    embp = jnp.pad(emb_table, ((0, 32 - VSZ), (0, 0)))
    b1r = b1[None, :]
    b2r = b2[None, :]
    bcr = bc[None, :]

    deg2 = _deg_kernel(dst)
    dega = deg2[0].reshape(NP, 1)
    degb = deg2[1].reshape(NP, 1)

    hs1 = _hs1_call(xf, dega, degb, embp, W1)
    s1 = _msg_kernel(hs1, src, dst)
    hs2 = _layer2_call(s1, hs1, dega, degb, b1r, W2)
    s2 = _msg_kernel(hs2, src, dst)
    gsum, cnt = _pool_call(s2, hs2, dega, degb, b2r, batchf)
    return _cls_call(gsum, cnt, Wc, bcr)
